# Initial kernel scaffold; baseline (speedup 1.0000x reference)
import jax
import jax.numpy as jnp
from jax.experimental import pallas as pl


def kernel(pos, batch, c1_W1, c1_b1, c1_W2, c1_b2, c2_W1, c2_b1, c2_W2, c2_b2, c3_W1, c3_b1, c3_W2, c3_b2, s_W1, s_b1, s_W2, s_b2, e_W1, e_b1, e_W2, e_b2):
    raise NotImplementedError("write your pallas kernel here")


# speedup vs baseline: 4.3762x; 4.3762x over previous
import functools

import jax
import jax.numpy as jnp
import numpy as np
from jax import lax
from jax.experimental import pallas as pl
from jax.experimental.pallas import tpu as pltpu
from jax.experimental.pallas import tpu_sc as plsc

N = 512
NPAIR = (N * (N - 1)) // 2
NWORK = 32
PW = NPAIR // NWORK
PW_PAD = 4096
WIN_ROWS = 128


def _relu(v):
    return jnp.maximum(v, 0.0)


def _bfr(v):
    return v.astype(jnp.bfloat16).astype(jnp.float32)


_TI, _TJ = np.triu_indices(N, k=1)
_FLAT_IDX = (_TI * N + _TJ).astype(np.int32)
_W_START = np.minimum(_TI[np.arange(NWORK) * PW], N - WIN_ROWS).astype(np.int32)
_IDX_LOCAL = np.zeros((NWORK, PW_PAD), np.int32)
for _w in range(NWORK):
    _seg = _FLAT_IDX[_w * PW:(_w + 1) * PW] - _W_START[_w] * N
    assert _seg.min() >= 0 and _seg.max() < WIN_ROWS * N
    _IDX_LOCAL[_w, :PW] = _seg


def _tc_body(pos_ref,
             c1w1, c1b1, c1w2, c1b2,
             c2w1, c2b1, c2w2, c2b2,
             c3w1, c3b1, c3w2, c3b2,
             sw1, sb1, sw2, sb2,
             ew1, eb1, ew2t, eb2,
             m_ref, a_ref, b_ref):
    x0 = pos_ref[...]

    iota_j = lax.broadcasted_iota(jnp.int32, (N, N), 1)

    def edge_conv(x, w1_ref, b1_ref, w2_ref, b2_ref):
        W1 = w1_ref[...]
        b1 = b1_ref[...]
        W2 = w2_ref[...]
        b2 = b2_ref[...]
        xt = x.T
        sq_col = jnp.sum(x * x, axis=1, keepdims=True)
        sq_row = sq_col.T
        if x.shape[1] <= 4:
            xxt = x[:, 0:1] * xt[0:1, :]
            for k in range(1, x.shape[1]):
                xxt = xxt + x[:, k:k + 1] * xt[k:k + 1, :]
        else:
            xxt = lax.dot_general(x, x, (((1,), (1,)), ((), ())),
                                  preferred_element_type=jnp.float32,
                                  precision=lax.Precision.HIGHEST)
        dist = sq_col + sq_row - 2.0 * xxt
        acc = None
        for _ in range(8):
            mmin = jnp.min(dist, axis=1, keepdims=True)
            am = jnp.min(jnp.where(dist == mmin, iota_j, N),
                         axis=1, keepdims=True)
            oneh = iota_j == am
            xj = lax.dot_general(oneh.astype(jnp.float32), x,
                                 (((1,), (0,)), ((), ())),
                                 preferred_element_type=jnp.float32,
                                 precision=lax.Precision.HIGHEST)
            e = jnp.concatenate([x, xj - x], axis=1)
            h1 = _relu(jnp.dot(_bfr(e), W1, preferred_element_type=jnp.float32,
                               precision=lax.Precision.HIGHEST) + b1)
            h2 = _relu(jnp.dot(_bfr(h1), W2, preferred_element_type=jnp.float32,
                               precision=lax.Precision.HIGHEST) + b2)
            acc = h2 if acc is None else jnp.maximum(acc, h2)
            dist = jnp.where(oneh, jnp.inf, dist)
        return acc

    x1 = edge_conv(x0, c1w1, c1b1, c1w2, c1b2)
    x2 = edge_conv(x1, c2w1, c2b1, c2w2, c2b2)
    x3 = edge_conv(x2, c3w1, c3b1, c3w2, c3b2)

    xc = jnp.concatenate([x1, x2, x3], axis=1)
    s1 = _relu(jnp.dot(_bfr(xc), sw1[...], preferred_element_type=jnp.float32,
                       precision=lax.Precision.HIGHEST) + sb1[...])
    sh = _relu(jnp.dot(_bfr(s1), sw2[...], preferred_element_type=jnp.float32,
                       precision=lax.Precision.HIGHEST) + sb2[...])
    shb = _bfr(sh)
    g = jnp.max(shb, axis=0, keepdims=True)

    eW1 = ew1[...]
    cvec = (lax.dot_general(g, eW1[256:384], (((1,), (0,)), ((), ())),
                            preferred_element_type=jnp.float32,
                            precision=lax.Precision.HIGHEST)
            + eb1[...])
    a_ref[...] = jnp.dot(shb, eW1[0:128],
                         preferred_element_type=jnp.float32,
                         precision=lax.Precision.HIGHEST) + cvec
    b_ref[...] = jnp.dot(shb, eW1[128:256],
                         preferred_element_type=jnp.float32,
                         precision=lax.Precision.HIGHEST)

    w2r = ew2t[...]
    b2s = eb2[0, 0]

    def pair_block(ib, carry):
        i0 = ib * 8
        ai = a_ref[pl.ds(i0, 8), :]
        bt = b_ref[...]
        t = _relu(ai[:, None, :] + bt[None, :, :])
        t = t.astype(jnp.bfloat16).astype(jnp.float32)
        mb = jnp.sum(t * w2r[None, :, :], axis=2)
        m_ref[pl.ds(i0, 8), :] = mb + b2s
        return carry

    lax.fori_loop(0, N // 8, pair_block, 0)


def _pair_matrix(pos,
                 c1_W1, c1_b1, c1_W2, c1_b2,
                 c2_W1, c2_b1, c2_W2, c2_b2,
                 c3_W1, c3_b1, c3_W2, c3_b2,
                 s_W1, s_b1, s_W2, s_b2,
                 e_W1, e_b1, e_W2, e_b2):
    r2 = lambda b: b.reshape(1, -1)
    return pl.pallas_call(
        _tc_body,
        out_shape=jax.ShapeDtypeStruct((N, N), jnp.float32),
        scratch_shapes=[pltpu.VMEM((N, 128), jnp.float32),
                        pltpu.VMEM((N, 128), jnp.float32)],
    )(pos,
      c1_W1, r2(c1_b1), c1_W2, r2(c1_b2),
      c2_W1, r2(c2_b1), c2_W2, r2(c2_b2),
      c3_W1, r2(c3_b1), c3_W2, r2(c3_b2),
      s_W1, r2(s_b1), s_W2, r2(s_b2),
      e_W1, r2(e_b1), e_W2.reshape(1, 128), e_b2.reshape(1, 1))


@functools.lru_cache(maxsize=1)
def _sc_pair_gather_fn():
    mesh = plsc.VectorSubcoreMesh(core_axis_name="c", subcore_axis_name="s")
    return functools.partial(
        pl.kernel, mesh=mesh,
        compiler_params=pltpu.CompilerParams(needs_layout_passes=False),
        out_type=(jax.ShapeDtypeStruct((NPAIR,), jnp.float32),
                  jax.ShapeDtypeStruct((NPAIR,), jnp.float32)),
        scratch_types=[pltpu.VMEM((WIN_ROWS * N,), jnp.float32),
                       pltpu.VMEM((PW_PAD,), jnp.int32),
                       pltpu.VMEM((PW_PAD,), jnp.float32),
                       pltpu.VMEM((PW_PAD,), jnp.float32)])(_sc_pair_body)


def _sc_pair_body(m_hbm, idx_hbm, probs_hbm, logits_hbm,
                  win_v, idx_v, pr_v, lg_v):
    wid = lax.axis_index("s") * 2 + lax.axis_index("c")
    woff = jnp.int32(0)
    for w in range(NWORK):
        woff = jnp.where(wid == w, jnp.int32(int(_W_START[w]) * N), woff)
    woff = pl.multiple_of(woff, 512)
    pltpu.sync_copy(m_hbm.at[pl.ds(woff, WIN_ROWS * N)], win_v)
    pltpu.sync_copy(idx_hbm.at[wid], idx_v)

    def chunk(ci, carry):
        iv = idx_v[pl.ds(ci * 16, 16)]
        vals = plsc.load_gather(win_v, [iv])
        lg_v[pl.ds(ci * 16, 16)] = vals
        pr_v[pl.ds(ci * 16, 16)] = 1.0 / (1.0 + jnp.exp(-vals))
        return carry

    lax.fori_loop(0, PW_PAD // 16, chunk, 0)
    base = pl.multiple_of(wid * PW, 8)
    pltpu.sync_copy(lg_v.at[pl.ds(0, PW)], logits_hbm.at[pl.ds(base, PW)])
    pltpu.sync_copy(pr_v.at[pl.ds(0, PW)], probs_hbm.at[pl.ds(base, PW)])


def kernel(pos, batch,
           c1_W1, c1_b1, c1_W2, c1_b2,
           c2_W1, c2_b1, c2_W2, c2_b2,
           c3_W1, c3_b1, c3_W2, c3_b2,
           s_W1, s_b1, s_W2, s_b2,
           e_W1, e_b1, e_W2, e_b2):
    m = _pair_matrix(pos,
                     c1_W1, c1_b1, c1_W2, c1_b2,
                     c2_W1, c2_b1, c2_W2, c2_b2,
                     c3_W1, c3_b1, c3_W2, c3_b2,
                     s_W1, s_b1, s_W2, s_b2,
                     e_W1, e_b1, e_W2, e_b2)
    idx = jnp.asarray(_IDX_LOCAL)
    probs, logits = _sc_pair_gather_fn()(m.reshape(N * N), idx)
    return probs.reshape(1, NPAIR), logits.reshape(1, NPAIR)
